# baseline (speedup 1.0000x reference)
import functools

import jax
import jax.numpy as jnp
from jax import lax
from jax.experimental import pallas as pl
from jax.experimental.pallas import tpu as pltpu
from jax.experimental.pallas import tpu_sc as plsc

_CELL_ANGLE = 0.087890625
_CELL_DEPTH = 0.015625
_NUM_PRED = 64
_BBOX = 7
_ANCHOR_W = 3.9
_ANCHOR_L = 1.6
_L_XY, _L_WL, _L_ROT, _L_OBJ, _L_NOOBJ = 10.0, 10.0, 20.0, 20.0, 1.0
_HALF_SPAN = 180.0

_NF = 16
_NLAB = 32
_NVAL = _NLAB * _BBOX
_ROWS_PER_FRAME = _NUM_PRED * _BBOX * 32


def _sc_gather_call(preds_flat, labels_t):
    mesh = plsc.VectorSubcoreMesh(core_axis_name="c", subcore_axis_name="s")
    half = _NVAL // 2

    @functools.partial(
        pl.kernel, mesh=mesh,
        out_type=jax.ShapeDtypeStruct((_NF, _NVAL, 128), jnp.float32),
        scratch_types=[
            pltpu.VMEM((_BBOX, _NLAB), jnp.float32),
            pltpu.VMEM((half,), jnp.int32),
            pltpu.VMEM((half,), jnp.int32),
            pltpu.VMEM((half, 128), jnp.float32),
            pltpu.VMEM((half, 128), jnp.float32),
            pltpu.SemaphoreType.DMA,
        ],
    )
    def sc_kernel(preds_hbm, lab_hbm, out_hbm,
                  lab_v, idx_a, idx_b, rows_a, rows_b, sem):
        wid = lax.axis_index("s") * 2 + lax.axis_index("c")

        @pl.when(wid < _NF)
        def _():
            f = wid
            pltpu.sync_copy(lab_hbm.at[f], lab_v)
            base = f * _ROWS_PER_FRAME
            for c in range(_NLAB // 16):
                l0 = lab_v[0, pl.ds(c * 16, 16)]
                l1 = lab_v[1, pl.ds(c * 16, 16)]
                ii = ((l0 + _HALF_SPAN) / _CELL_ANGLE).astype(jnp.int32)
                jj = (l1 / _CELL_DEPTH).astype(jnp.int32)
                rowbase = base + (ii >> 7)
                for b in range(_BBOX):
                    row = ((jj * _BBOX + b) << 5) + rowbase
                    pos = b * _NLAB + c * 16
                    if pos < half:
                        idx_a[pl.ds(pos, 16)] = row
                    else:
                        idx_b[pl.ds(pos - half, 16)] = row
            cp_a = pltpu.async_copy(preds_hbm.at[idx_a], rows_a, sem)
            cp_b = pltpu.async_copy(preds_hbm.at[idx_b], rows_b, sem)
            cp_a.wait()
            cp_b.wait()
            pltpu.sync_copy(rows_a, out_hbm.at[f, pl.ds(0, half)])
            pltpu.sync_copy(rows_b, out_hbm.at[f, pl.ds(half, half)])

    return sc_kernel(preds_flat, labels_t)


def _softplus(x):
    return jnp.maximum(x, 0.0) + jnp.log(1.0 + jnp.exp(-jnp.abs(x)))


def _tc_body(obj_ref, lab_ref, g_ref, out_ref):
    f = pl.program_id(0)
    nf = pl.num_programs(0)

    row_i = lax.broadcasted_iota(jnp.int32, (8, 128), 0)
    lane_i = lax.broadcasted_iota(jnp.int32, (8, 128), 1)

    def slot(i, v):
        return jnp.where((row_i == 0) & (lane_i == i), v, 0.0)

    @pl.when(f == 0)
    def _():
        lab = lab_ref[...]
        plab0 = lab[:, :, 0] + _HALF_SPAN
        iif = jnp.floor(plab0 / _CELL_ANGLE)
        jjf = jnp.floor(lab[:, :, 1] / _CELL_DEPTH)
        eq = ((iif[:, :, None] == iif[:, None, :])
              & (jjf[:, :, None] == jjf[:, None, :]))
        k_row = lax.broadcasted_iota(jnp.int32, (_NF, _NLAB, _NLAB), 1)
        k_col = lax.broadcasted_iota(jnp.int32, (_NF, _NLAB, _NLAB), 2)
        killed = jnp.any(eq & (k_col > k_row), axis=2)
        has_labels = lab[:, 0, 6] >= 0.0
        live = jnp.logical_and(jnp.logical_not(killed), has_labels[:, None])
        livef = live.astype(jnp.float32)

        rows = g_ref[...].reshape(_NF, _BBOX, _NLAB, 128)
        lane = iif.astype(jnp.int32) & 127
        lane_oh = (lax.broadcasted_iota(jnp.int32, (_NF, _NLAB, 128), 2)
                   == lane[:, :, None])

        def sel(b):
            return jnp.sum(jnp.where(lane_oh, rows[:, b], 0.0), axis=2)

        g0, g1, g2, g3, g4, g5, g6 = (sel(b) for b in range(_BBOX))
        tx = plab0 / _CELL_ANGLE - iif
        ty = lab[:, :, 1] / _CELL_DEPTH - jjf
        tw = jnp.log(lab[:, :, 2] / _ANCHOR_W + 1e-16)
        tl = jnp.log(lab[:, :, 3] / _ANCHOR_L + 1e-16)

        sx = jax.nn.sigmoid(g0)
        sy = jax.nn.sigmoid(g1)
        l_xy = jnp.sum(livef * ((sx - tx) ** 2 + (sy - ty) ** 2))
        l_wl = jnp.sum(livef * ((g2 - tw) ** 2 + (g3 - tl) ** 2))
        l_rot = jnp.sum(livef * ((jnp.tanh(g4) - lab[:, :, 4]) ** 2
                                 + (jnp.tanh(g5) - lab[:, :, 5]) ** 2))
        pobj = jax.nn.sigmoid(g6)
        l_obj = jnp.sum(livef * (-jnp.maximum(jnp.log(pobj), -100.0)))
        noobj_corr = jnp.sum(livef * _softplus(g6))
        out_ref[...] = (slot(1, l_xy) + slot(2, l_wl) + slot(3, l_rot)
                        + slot(4, l_obj) + slot(5, -noobj_corr))

    x = obj_ref[0, :, 0]
    dense = jnp.sum(_softplus(x))
    out_ref[...] = out_ref[...] + slot(5, dense)

    @pl.when(f == nf - 1)
    def _():
        acc = out_ref[...]
        w = (slot(1, _L_XY) + slot(2, _L_WL) + slot(3, _L_ROT)
             + slot(4, _L_OBJ) + slot(5, _L_NOOBJ))
        out_ref[...] = acc + slot(0, jnp.sum(acc * w))


def kernel(preds, labels):
    nf, nchan, row_size = preds.shape
    preds5 = preds.reshape(nf, _NUM_PRED, _BBOX, row_size // 128, 128)
    preds_flat = preds.reshape(nf * nchan * (row_size // 128), 128)
    labels_t = jnp.transpose(labels, (0, 2, 1))
    gathered = _sc_gather_call(preds_flat, labels_t)
    out = pl.pallas_call(
        _tc_body,
        grid=(nf,),
        in_specs=[
            pl.BlockSpec((1, _NUM_PRED, 1, row_size // 128, 128),
                         lambda f: (f, 0, 6, 0, 0)),
            pl.BlockSpec((nf, _NLAB, _BBOX), lambda f: (0, 0, 0)),
            pl.BlockSpec((nf, _NVAL, 128), lambda f: (0, 0, 0)),
        ],
        out_specs=pl.BlockSpec((8, 128), lambda f: (0, 0)),
        out_shape=jax.ShapeDtypeStruct((8, 128), jnp.float32),
    )(preds5, labels, gathered)
    return (out[0, 0], out[0, 1], out[0, 2], out[0, 3], out[0, 4], out[0, 5])
